# baseline (speedup 1.0000x reference)
import jax
import jax.numpy as jnp
from jax import lax
from jax.experimental import pallas as pl
from jax.experimental.pallas import tpu as pltpu, tpu_sc as plsc

NC = 2
NS = 16
NW = NC * NS
D = 64
CHUNK = 400
NBUF = 4


def _gather_body(ids_hbm, table_hbm, out_hbm, idx_v, *rest):
    rows = rest[:NBUF]
    gsems = rest[NBUF:2 * NBUF]
    ssems = rest[2 * NBUF:]
    b_per_w = ids_hbm.shape[1]
    n_chunks = b_per_w // CHUNK
    wid = lax.axis_index("s") * NC + lax.axis_index("c")
    base = wid * b_per_w

    pltpu.sync_copy(ids_hbm.at[wid], idx_v)

    def idx_slice(c):
        return idx_v.at[pl.ds(c * CHUNK, CHUNK)]

    def out_slice(c):
        return out_hbm.at[pl.ds(base + c * CHUNK, CHUNK)]

    LOOK = NBUF // 2

    def wait_gather(c, b):
        pltpu.make_async_copy(table_hbm.at[idx_slice(c)], rows[b], gsems[b]).wait()

    def wait_store(c, b):
        pltpu.make_async_copy(rows[b], out_slice(c), ssems[b]).wait()

    for b in range(LOOK):
        pltpu.async_copy(table_hbm.at[idx_slice(b)], rows[b], gsems[b])
    for c in range(LOOK):
        b = c % NBUF
        wait_gather(c, b)
        pltpu.async_copy(rows[b], out_slice(c), ssems[b])
        b2 = (c + LOOK) % NBUF
        pltpu.async_copy(table_hbm.at[idx_slice(c + LOOK)], rows[b2], gsems[b2])

    @pl.loop(LOOK, n_chunks - LOOK, step=NBUF)
    def _(j):
        for u in range(NBUF):
            c = j + u
            b = (LOOK + u) % NBUF
            wait_gather(c, b)
            pltpu.async_copy(rows[b], out_slice(c), ssems[b])
            b2 = (b + LOOK) % NBUF
            wait_store(c - LOOK, b2)
            pltpu.async_copy(table_hbm.at[idx_slice(c + LOOK)], rows[b2], gsems[b2])

    for u in range(LOOK):
        c = n_chunks - LOOK + u
        b = c % NBUF
        wait_gather(c, b)
        pltpu.async_copy(rows[b], out_slice(c), ssems[b])
    for u in range(NBUF):
        c = n_chunks - NBUF + u
        wait_store(c, c % NBUF)


def kernel(input_ids, attention_mask, table):
    del attention_mask
    batch, hist = input_ids.shape
    b_total = batch * hist
    assert b_total % (NW * CHUNK) == 0
    b_per_w = b_total // NW
    ids2 = input_ids.reshape(NW, b_per_w).astype(jnp.int32)

    run = pl.kernel(
        _gather_body,
        out_type=jax.ShapeDtypeStruct((b_total, D), jnp.float32),
        mesh=plsc.VectorSubcoreMesh(
            core_axis_name="c", subcore_axis_name="s",
            num_cores=NC, num_subcores=NS,
        ),
        compiler_params=pltpu.CompilerParams(use_tc_tiling_on_sc=False),
        scratch_types=(
            [pltpu.VMEM((b_per_w,), jnp.int32)]
            + [pltpu.VMEM((CHUNK, D), jnp.float32) for _ in range(NBUF)]
            + [pltpu.SemaphoreType.DMA for _ in range(2 * NBUF)]
        ),
    )
    out = run(ids2, table)
    return out.reshape(batch, hist, D)


# speedup vs baseline: 1.0050x; 1.0050x over previous
import jax
import jax.numpy as jnp
from jax import lax
from jax.experimental import pallas as pl
from jax.experimental.pallas import tpu as pltpu, tpu_sc as plsc

NC = 2
NS = 16
NW = NC * NS
D = 64
HIST = 200
NBUF = 4


def _gather_body(ids_hbm, table_hbm, out_hbm, idx_v, *rest):
    rows = rest[:NBUF]
    gsems = rest[NBUF:2 * NBUF]
    ssems = rest[2 * NBUF:]
    n_chunks = out_hbm.shape[0] // NW
    b_per_w = n_chunks * HIST
    wid = lax.axis_index("s") * NC + lax.axis_index("c")
    base = wid * n_chunks

    pltpu.sync_copy(ids_hbm.at[pl.ds(wid * b_per_w, b_per_w)], idx_v)

    def idx_slice(c):
        return idx_v.at[pl.ds(c * HIST, HIST)]

    def out_slice(c):
        return out_hbm.at[base + c]

    LOOK = NBUF // 2

    def wait_gather(c, b):
        pltpu.make_async_copy(table_hbm.at[idx_slice(c)], rows[b], gsems[b]).wait()

    def wait_store(c, b):
        pltpu.make_async_copy(rows[b], out_slice(c), ssems[b]).wait()

    for b in range(LOOK):
        pltpu.async_copy(table_hbm.at[idx_slice(b)], rows[b], gsems[b])
    for c in range(LOOK):
        b = c % NBUF
        wait_gather(c, b)
        pltpu.async_copy(rows[b], out_slice(c), ssems[b])
        b2 = (c + LOOK) % NBUF
        pltpu.async_copy(table_hbm.at[idx_slice(c + LOOK)], rows[b2], gsems[b2])

    @pl.loop(LOOK, n_chunks - LOOK, step=NBUF)
    def _(j):
        for u in range(NBUF):
            c = j + u
            b = (LOOK + u) % NBUF
            wait_gather(c, b)
            pltpu.async_copy(rows[b], out_slice(c), ssems[b])
            b2 = (b + LOOK) % NBUF
            wait_store(c - LOOK, b2)
            pltpu.async_copy(table_hbm.at[idx_slice(c + LOOK)], rows[b2], gsems[b2])

    for u in range(LOOK):
        c = n_chunks - LOOK + u
        b = c % NBUF
        wait_gather(c, b)
        pltpu.async_copy(rows[b], out_slice(c), ssems[b])
    for u in range(NBUF):
        c = n_chunks - NBUF + u
        wait_store(c, c % NBUF)


def kernel(input_ids, attention_mask, table):
    del attention_mask
    batch, hist = input_ids.shape
    assert hist == HIST and batch % NW == 0
    ids1d = input_ids.reshape(batch * hist).astype(jnp.int32)
    b_per_w = (batch // NW) * hist

    run = pl.kernel(
        _gather_body,
        out_type=jax.ShapeDtypeStruct((batch, hist, D), jnp.float32),
        mesh=plsc.VectorSubcoreMesh(
            core_axis_name="c", subcore_axis_name="s",
            num_cores=NC, num_subcores=NS,
        ),
        compiler_params=pltpu.CompilerParams(use_tc_tiling_on_sc=False),
        scratch_types=(
            [pltpu.VMEM((b_per_w,), jnp.int32)]
            + [pltpu.VMEM((HIST, D), jnp.float32) for _ in range(NBUF)]
            + [pltpu.SemaphoreType.DMA for _ in range(2 * NBUF)]
        ),
    )
    return run(ids1d, table)
